# baseline (speedup 1.0000x reference)
import functools

import jax
import jax.numpy as jnp
import numpy as np
from jax import lax
from jax.experimental import pallas as pl
from jax.experimental.pallas import tpu as pltpu
from jax.experimental.pallas import tpu_sc as plsc

N = 10000
E = 320000
IN_DIM = 128
HID = 512
LAT = 64
C = 50
TAU = 1.0

NC = 2
NS = 16
NW = NC * NS
CHUNK = 128
CH = 80
NP_ = 10240
STRIPE = NP_ // NS

CH0 = 120
CH1 = 40
CHMAX = max(CH0, CH1)
TOTCH = NS * (CH0 + CH1)
RPAD = TOTCH + CHMAX
EPAD = RPAD * CHUNK


def _sc_deg_body(dst_hbm, w_hbm, zeros_hbm, out_hbm, dst_v, w_v, acc, sem):
    c = lax.axis_index("c")
    s = lax.axis_index("s")
    wid = s * NC + c
    row0 = s * STRIPE

    pltpu.sync_copy(zeros_hbm.at[pl.ds(row0, STRIPE)], acc.at[pl.ds(row0, STRIPE)])
    pltpu.sync_copy(dst_hbm.at[pl.ds(wid * CH, CH)], dst_v)
    pltpu.sync_copy(w_hbm.at[pl.ds(wid * CH, CH)], w_v)
    plsc.subcore_barrier()

    GRP = 8

    def outer(i, _):
        j0 = i * GRP
        for b in range(GRP):
            pltpu.async_copy(w_v.at[j0 + b], acc.at[dst_v.at[j0 + b]], sem,
                             add=True)
        for b in range(GRP):
            pltpu.make_async_copy(w_v.at[j0 + b], acc.at[dst_v.at[j0 + b]],
                                  sem).wait()
        return 0

    lax.fori_loop(0, CH // GRP, outer, 0)
    plsc.subcore_barrier()
    pltpu.sync_copy(acc.at[pl.ds(row0, STRIPE)], out_hbm.at[c, pl.ds(row0, STRIPE)])


FW = 64
NBUF = 4
FBUF = 2
SD = 2


PC = 20


def _sc_agg_body(src_hbm, dst_hbm, w_hbm, xlo_hbm, xhi_hbm, out_hbm,
                 src_v, dst_v, w_v, rows16_v, fbuf_v, acc, gsem, ssem):
    c = lax.axis_index("c")
    s = lax.axis_index("s")
    row0 = s * STRIPE

    ch_c = jnp.where(c == 0, CH0, CH1)
    crow0 = jnp.where(c == 0, s * CH0, NS * CH0 + s * CH1)

    def ldbody(v, _):
        hsl = pl.ds(crow0 + v * PC, PC)
        vsl = pl.ds(v * PC, PC)
        pltpu.sync_copy(src_hbm.at[hsl], src_v.at[vsl])
        pltpu.sync_copy(dst_hbm.at[hsl], dst_v.at[vsl])
        pltpu.sync_copy(w_hbm.at[hsl], w_v.at[vsl])
        return 0

    lax.fori_loop(0, ch_c // PC, ldbody, 0)

    zero16 = jnp.zeros((16,), jnp.float32)

    for h, x_hbm in enumerate((xlo_hbm, xhi_hbm)):
        def zbody(g, _):
            fbuf_v[0, g >> 2, pl.ds((g & 3) * 16, 16)] = zero16
            return 0

        lax.fori_loop(0, CHUNK * (FW // 16), zbody, 0)
        for zi in range(STRIPE // CHUNK):
            pltpu.sync_copy(fbuf_v.at[0],
                            acc.at[pl.ds(row0 + zi * CHUNK, CHUNK)])
        plsc.subcore_barrier()

        for p in range(NBUF - 1):
            pltpu.async_copy(x_hbm.at[src_v.at[p]], rows16_v.at[p], gsem)

        def outer(i, _):
            j0 = i * NBUF
            for b in range(NBUF):
                j = j0 + b

                @pl.when(j + NBUF - 1 < ch_c)
                def _start_next():
                    pltpu.async_copy(x_hbm.at[src_v.at[j + NBUF - 1]],
                                     rows16_v.at[(b - 1) % NBUF], gsem)

                pltpu.make_async_copy(x_hbm.at[src_v.at[j]], rows16_v.at[b],
                                      gsem).wait()

                fb = b % FBUF

                @pl.when(j >= SD)
                def _drain_scatter():
                    pltpu.make_async_copy(fbuf_v.at[fb],
                                          acc.at[dst_v.at[j - SD]], ssem).wait()

                def gbody(g, _):
                    wv = w_v[j, pl.ds(g * 16, 16)]
                    for i16 in range(16):
                        r = g * 16 + i16
                        wb = jnp.full((16,), wv[i16])
                        for k in range(FW // 32):
                            x32 = rows16_v[b, r, pl.ds(k * 32, 32)]
                            lo, hi = plsc.unpack(
                                x32, format=plsc.PackFormat.INTERLEAVED)
                            fbuf_v[fb, r, pl.ds(k * 32, 16)] = lo * wb
                            fbuf_v[fb, r, pl.ds(k * 32 + 16, 16)] = hi * wb
                    return 0

                lax.fori_loop(0, CHUNK // 16, gbody, 0)
                pltpu.async_copy(fbuf_v.at[fb], acc.at[dst_v.at[j]], ssem,
                                 add=True)
            return 0

        lax.fori_loop(0, ch_c // NBUF, outer, 0)
        for d in range(SD):
            j = ch_c - SD + d
            pltpu.make_async_copy(fbuf_v.at[(d - SD) % FBUF],
                                  acc.at[dst_v.at[j]], ssem).wait()
        plsc.subcore_barrier()
        pltpu.sync_copy(acc.at[pl.ds(row0, STRIPE)],
                        out_hbm.at[c, h, pl.ds(row0, STRIPE)])


@functools.cache
def _sc_kernels():
    mesh = plsc.VectorSubcoreMesh(core_axis_name="c", subcore_axis_name="s")
    params = pltpu.CompilerParams(use_tc_tiling_on_sc=False,
                                  needs_layout_passes=False)
    sc_deg = functools.partial(
        pl.kernel,
        out_type=jax.ShapeDtypeStruct((NC, NP_), jnp.float32),
        mesh=mesh,
        compiler_params=params,
        scratch_types=[
            pltpu.VMEM((CH, CHUNK), jnp.int32),
            pltpu.VMEM((CH, CHUNK), jnp.float32),
            pltpu.VMEM_SHARED((NP_,), jnp.float32),
            pltpu.SemaphoreType.DMA,
        ],
    )(_sc_deg_body)
    sc_agg = functools.partial(
        pl.kernel,
        out_type=jax.ShapeDtypeStruct((NC, 2, NP_, FW), jnp.float32),
        mesh=mesh,
        compiler_params=params,
        scratch_types=[
            pltpu.VMEM((CHMAX, CHUNK), jnp.int32),
            pltpu.VMEM((CHMAX, CHUNK), jnp.int32),
            pltpu.VMEM((CHMAX, CHUNK), jnp.float32),
            pltpu.VMEM((NBUF, CHUNK, FW), jnp.bfloat16),
            pltpu.VMEM((FBUF, CHUNK, FW), jnp.float32),
            pltpu.VMEM_SHARED((NP_, FW), jnp.float32),
            pltpu.SemaphoreType.DMA,
            pltpu.SemaphoreType.DMA,
        ],
    )(_sc_agg_body)
    return sc_deg, sc_agg


_BLK = 1000


def _tc_prep_body(deg_ref, y_ref, dinv_ref, ylo_ref, yhi_ref):
    d = deg_ref[0] + deg_ref[1] + 1.0
    dinv = jnp.where(d > 0, lax.rsqrt(d), 0.0)
    dinv_ref[...] = dinv
    ys = y_ref[...] * dinv
    ylo_ref[...] = ys[:, :FW]
    yhi_ref[...] = ys[:, FW:]


def _tc_prep(deg3, Y):
    return pl.pallas_call(
        _tc_prep_body,
        grid=(N // _BLK,),
        in_specs=[
            pl.BlockSpec((2, _BLK, 1), lambda i: (0, i, 0)),
            pl.BlockSpec((_BLK, IN_DIM), lambda i: (i, 0)),
        ],
        out_specs=[
            pl.BlockSpec((_BLK, 1), lambda i: (i, 0)),
            pl.BlockSpec((_BLK, FW), lambda i: (i, 0)),
            pl.BlockSpec((_BLK, FW), lambda i: (i, 0)),
        ],
        out_shape=[
            jax.ShapeDtypeStruct((N, 1), jnp.float32),
            jax.ShapeDtypeStruct((N, FW), jnp.float32),
            jax.ShapeDtypeStruct((N, FW), jnp.float32),
        ],
    )(deg3, Y)


def _combine_u(u_ref):
    return jnp.concatenate([u_ref[0, 0] + u_ref[1, 0],
                            u_ref[0, 1] + u_ref[1, 1]], axis=1)


def _tc_enc_body(u_ref, ylo_ref, yhi_ref, dinv_ref, w1_ref, b1_ref, wcat_ref,
                 hlo_ref, hhi_ref):
    dinv = dinv_ref[...]
    ys = jnp.concatenate([ylo_ref[...], yhi_ref[...]], axis=1)
    ya = (_combine_u(u_ref) + ys) * dinv
    h = jnp.dot(ya, w1_ref[...], preferred_element_type=jnp.float32) + b1_ref[...]
    h = jnp.maximum(h, 0.0)
    hm = jnp.dot(h, wcat_ref[...], preferred_element_type=jnp.float32)
    hs = hm * dinv
    hlo_ref[...] = hs[:, :FW]
    hhi_ref[...] = hs[:, FW:]


def _tc_enc(U1, Ylo, Yhi, dinv, W1, b1r, Wcat):
    return pl.pallas_call(
        _tc_enc_body,
        grid=(N // _BLK,),
        in_specs=[
            pl.BlockSpec((2, 2, _BLK, FW), lambda i: (0, 0, i, 0)),
            pl.BlockSpec((_BLK, FW), lambda i: (i, 0)),
            pl.BlockSpec((_BLK, FW), lambda i: (i, 0)),
            pl.BlockSpec((_BLK, 1), lambda i: (i, 0)),
            pl.BlockSpec((IN_DIM, HID), lambda i: (0, 0)),
            pl.BlockSpec((1, HID), lambda i: (0, 0)),
            pl.BlockSpec((HID, 2 * LAT), lambda i: (0, 0)),
        ],
        out_specs=[
            pl.BlockSpec((_BLK, FW), lambda i: (i, 0)),
            pl.BlockSpec((_BLK, FW), lambda i: (i, 0)),
        ],
        out_shape=[
            jax.ShapeDtypeStruct((N, FW), jnp.float32),
            jax.ShapeDtypeStruct((N, FW), jnp.float32),
        ],
    )(U1, Ylo, Yhi, dinv, W1, b1r, Wcat)


def _tc_dec_body(u_ref, hlo_ref, hhi_ref, dinv_ref, eps_ref, bmu_ref, blv_ref,
                 wd1_ref, bd1_ref, wd2_ref, bd2_ref, xref_ref,
                 yhat_ref, mu_ref, lv_ref, b_ref):
    dinv = dinv_ref[...]
    hs = jnp.concatenate([hlo_ref[...], hhi_ref[...]], axis=1)
    agg = (_combine_u(u_ref) + hs) * dinv
    mu = agg[:, :LAT] + bmu_ref[...]
    lv = jnp.clip(agg[:, LAT:] + blv_ref[...], -10.0, 10.0)
    z = mu + eps_ref[...] * jnp.exp(0.5 * lv)
    hd = jnp.dot(z, wd1_ref[...], preferred_element_type=jnp.float32) + bd1_ref[...]
    hd = jnp.maximum(hd, 0.0)
    logits = (jnp.dot(hd, wd2_ref[...], preferred_element_type=jnp.float32)
              + bd2_ref[...]) / TAU
    m = jnp.max(logits, axis=1, keepdims=True)
    e = jnp.exp(logits - m)
    bmat = e / jnp.sum(e, axis=1, keepdims=True)
    yhat_ref[...] = jnp.dot(bmat, xref_ref[...], preferred_element_type=jnp.float32)
    mu_ref[...] = mu
    lv_ref[...] = lv
    b_ref[...] = bmat[:, :C]


def _tc_dec(U2, Hlo, Hhi, dinv, eps, bmur, blvr, Wd1, bd1r, Wd2p, bd2p, Xrefp):
    CPAD = LAT
    return pl.pallas_call(
        _tc_dec_body,
        grid=(N // _BLK,),
        in_specs=[
            pl.BlockSpec((2, 2, _BLK, FW), lambda i: (0, 0, i, 0)),
            pl.BlockSpec((_BLK, FW), lambda i: (i, 0)),
            pl.BlockSpec((_BLK, FW), lambda i: (i, 0)),
            pl.BlockSpec((_BLK, 1), lambda i: (i, 0)),
            pl.BlockSpec((_BLK, LAT), lambda i: (i, 0)),
            pl.BlockSpec((1, LAT), lambda i: (0, 0)),
            pl.BlockSpec((1, LAT), lambda i: (0, 0)),
            pl.BlockSpec((LAT, HID), lambda i: (0, 0)),
            pl.BlockSpec((1, HID), lambda i: (0, 0)),
            pl.BlockSpec((HID, CPAD), lambda i: (0, 0)),
            pl.BlockSpec((1, CPAD), lambda i: (0, 0)),
            pl.BlockSpec((CPAD, IN_DIM), lambda i: (0, 0)),
        ],
        out_specs=[
            pl.BlockSpec((_BLK, IN_DIM), lambda i: (i, 0)),
            pl.BlockSpec((_BLK, LAT), lambda i: (i, 0)),
            pl.BlockSpec((_BLK, LAT), lambda i: (i, 0)),
            pl.BlockSpec((_BLK, C), lambda i: (i, 0)),
        ],
        out_shape=[
            jax.ShapeDtypeStruct((N, IN_DIM), jnp.float32),
            jax.ShapeDtypeStruct((N, LAT), jnp.float32),
            jax.ShapeDtypeStruct((N, LAT), jnp.float32),
            jax.ShapeDtypeStruct((N, C), jnp.float32),
        ],
    )(U2, Hlo, Hhi, dinv, eps, bmur, blvr, Wd1, bd1r, Wd2p, bd2p, Xrefp)


def kernel(Y, edge_index, edge_weight, X_ref, W1, b1, Wmu, bmu, Wlv, blv,
           Wd1, bd1, Wd2, bd2):
    f32 = jnp.float32
    src = edge_index[0].astype(jnp.int32)
    dst = edge_index[1].astype(jnp.int32)
    pad = EPAD - E
    zi = jnp.zeros((pad,), jnp.int32)
    srcF = jnp.concatenate([src, zi]).reshape(RPAD, CHUNK)
    dstF = jnp.concatenate([dst, zi]).reshape(RPAD, CHUNK)
    wF = jnp.concatenate([edge_weight, jnp.zeros((pad,), f32)]).reshape(RPAD, CHUNK)

    eps = jax.random.normal(jax.random.key(42), (N, LAT), dtype=f32)

    Wcat = jnp.concatenate([Wmu, Wlv], axis=1)
    Wd2p = jnp.concatenate([Wd2, jnp.zeros((HID, LAT - C), f32)], axis=1)
    bd2p = jnp.concatenate([bd2, jnp.full((LAT - C,), -1e30, f32)]).reshape(1, LAT)
    Xrefp = jnp.concatenate([X_ref, jnp.zeros((LAT - C, IN_DIM), f32)], axis=0)

    perm = np.concatenate([
        np.stack([np.arange(16) + g * 32, np.arange(16) + g * 32 + 16],
                 axis=1).ravel()
        for g in range(FW // 32)
    ])

    def bf16p(x):
        return x[:, perm].astype(jnp.bfloat16)

    sc_deg, sc_agg = _sc_kernels()
    deg2 = sc_deg(dstF, wF, jnp.zeros((NP_,), f32))
    dinv, Ylo, Yhi = _tc_prep(deg2.reshape(NC, NP_, 1), Y)
    U1 = sc_agg(srcF, dstF, wF, bf16p(Ylo), bf16p(Yhi))
    Hlo, Hhi = _tc_enc(U1, Ylo, Yhi, dinv, W1, b1.reshape(1, HID), Wcat)
    U2 = sc_agg(srcF, dstF, wF, bf16p(Hlo), bf16p(Hhi))
    Y_hat, mu, logvar, Bmat = _tc_dec(
        U2, Hlo, Hhi, dinv, eps, bmu.reshape(1, LAT), blv.reshape(1, LAT),
        Wd1, bd1.reshape(1, HID), Wd2p, bd2p, Xrefp)
    return (Y_hat, mu, logvar, Bmat)


# speedup vs baseline: 1.0296x; 1.0296x over previous
import functools

import jax
import jax.numpy as jnp
import numpy as np
from jax import lax
from jax.experimental import pallas as pl
from jax.experimental.pallas import tpu as pltpu
from jax.experimental.pallas import tpu_sc as plsc

N = 10000
E = 320000
IN_DIM = 128
HID = 512
LAT = 64
C = 50
TAU = 1.0

NC = 2
NS = 16
NW = NC * NS
CHUNK = 128
CH = 80
NP_ = 10240
STRIPE = NP_ // NS

CH0 = 120
CH1 = 39
CHMAX = max(CH0, CH1)
TOTCH = NS * (CH0 + CH1)
RPAD = TOTCH + CHMAX
EPAD = RPAD * CHUNK


def _sc_deg_body(dst_hbm, w_hbm, zeros_hbm, out_hbm, dst_v, w_v, acc, sem):
    c = lax.axis_index("c")
    s = lax.axis_index("s")
    wid = s * NC + c
    row0 = s * STRIPE

    pltpu.sync_copy(zeros_hbm.at[pl.ds(row0, STRIPE)], acc.at[pl.ds(row0, STRIPE)])
    pltpu.sync_copy(dst_hbm.at[pl.ds(wid * CH, CH)], dst_v)
    pltpu.sync_copy(w_hbm.at[pl.ds(wid * CH, CH)], w_v)
    plsc.subcore_barrier()

    GRP = 8

    def outer(i, _):
        j0 = i * GRP
        for b in range(GRP):
            pltpu.async_copy(w_v.at[j0 + b], acc.at[dst_v.at[j0 + b]], sem,
                             add=True)
        for b in range(GRP):
            pltpu.make_async_copy(w_v.at[j0 + b], acc.at[dst_v.at[j0 + b]],
                                  sem).wait()
        return 0

    lax.fori_loop(0, CH // GRP, outer, 0)
    plsc.subcore_barrier()
    pltpu.sync_copy(acc.at[pl.ds(row0, STRIPE)], out_hbm.at[c, pl.ds(row0, STRIPE)])


FW = 64
NBUF = 3
SD = 2


def _sc_agg_body(src_hbm, dst_hbm, w_hbm, xlo_hbm, xhi_hbm, zeros_hbm, out_hbm,
                 src_v, dst_v, w_v, rows16_v, fbuf_v, acc, gsem, ssem):
    c = lax.axis_index("c")
    s = lax.axis_index("s")
    row0 = s * STRIPE

    ch_c = jnp.where(c == 0, CH0, CH1)
    crow0 = jnp.where(c == 0, s * CH0, NS * CH0 + s * CH1)

    pltpu.sync_copy(src_hbm.at[pl.ds(crow0, CHMAX)], src_v)
    pltpu.sync_copy(dst_hbm.at[pl.ds(crow0, CHMAX)], dst_v)
    pltpu.sync_copy(w_hbm.at[pl.ds(crow0, CHMAX)], w_v)

    for h, x_hbm in enumerate((xlo_hbm, xhi_hbm)):
        pltpu.sync_copy(zeros_hbm.at[pl.ds(row0, STRIPE)],
                        acc.at[pl.ds(row0, STRIPE)])
        plsc.subcore_barrier()

        for p in range(NBUF - 1):
            pltpu.async_copy(x_hbm.at[src_v.at[p]], rows16_v.at[p], gsem)

        def outer(i, _):
            j0 = i * NBUF
            for b in range(NBUF):
                j = j0 + b

                @pl.when(j + NBUF - 1 < ch_c)
                def _start_next():
                    pltpu.async_copy(x_hbm.at[src_v.at[j + NBUF - 1]],
                                     rows16_v.at[(b - 1) % NBUF], gsem)

                pltpu.make_async_copy(x_hbm.at[src_v.at[j]], rows16_v.at[b],
                                      gsem).wait()

                def gbody(g, _):
                    wv = w_v[j, pl.ds(g * 16, 16)]
                    for i16 in range(16):
                        r = g * 16 + i16
                        wb = jnp.full((16,), wv[i16])
                        for k in range(FW // 32):
                            x32 = rows16_v[b, r, pl.ds(k * 32, 32)]
                            lo, hi = plsc.unpack(
                                x32, format=plsc.PackFormat.INTERLEAVED)
                            fbuf_v[b, r, pl.ds(k * 32, 16)] = lo * wb
                            fbuf_v[b, r, pl.ds(k * 32 + 16, 16)] = hi * wb
                    return 0

                lax.fori_loop(0, CHUNK // 16, gbody, 0)

                @pl.when(j >= SD)
                def _drain_scatter():
                    bb = (b - SD) % NBUF
                    pltpu.make_async_copy(fbuf_v.at[bb],
                                          acc.at[dst_v.at[j - SD]], ssem).wait()

                pltpu.async_copy(fbuf_v.at[b], acc.at[dst_v.at[j]], ssem,
                                 add=True)
            return 0

        lax.fori_loop(0, ch_c // NBUF, outer, 0)
        for d in range(SD):
            j = ch_c - SD + d
            pltpu.make_async_copy(fbuf_v.at[(d - SD) % NBUF],
                                  acc.at[dst_v.at[j]], ssem).wait()
        plsc.subcore_barrier()
        pltpu.sync_copy(acc.at[pl.ds(row0, STRIPE)],
                        out_hbm.at[c, h, pl.ds(row0, STRIPE)])


@functools.cache
def _sc_kernels():
    mesh = plsc.VectorSubcoreMesh(core_axis_name="c", subcore_axis_name="s")
    params = pltpu.CompilerParams(use_tc_tiling_on_sc=False,
                                  needs_layout_passes=False)
    sc_deg = functools.partial(
        pl.kernel,
        out_type=jax.ShapeDtypeStruct((NC, NP_), jnp.float32),
        mesh=mesh,
        compiler_params=params,
        scratch_types=[
            pltpu.VMEM((CH, CHUNK), jnp.int32),
            pltpu.VMEM((CH, CHUNK), jnp.float32),
            pltpu.VMEM_SHARED((NP_,), jnp.float32),
            pltpu.SemaphoreType.DMA,
        ],
    )(_sc_deg_body)
    sc_agg = functools.partial(
        pl.kernel,
        out_type=jax.ShapeDtypeStruct((NC, 2, NP_, FW), jnp.float32),
        mesh=mesh,
        compiler_params=params,
        scratch_types=[
            pltpu.VMEM((CHMAX, CHUNK), jnp.int32),
            pltpu.VMEM((CHMAX, CHUNK), jnp.int32),
            pltpu.VMEM((CHMAX, CHUNK), jnp.float32),
            pltpu.VMEM((NBUF, CHUNK, FW), jnp.bfloat16),
            pltpu.VMEM((NBUF, CHUNK, FW), jnp.float32),
            pltpu.VMEM_SHARED((NP_, FW), jnp.float32),
            pltpu.SemaphoreType.DMA,
            pltpu.SemaphoreType.DMA,
        ],
    )(_sc_agg_body)
    return sc_deg, sc_agg


_BLK = 1000


def _tc_prep_body(deg_ref, y_ref, dinv_ref, ylo_ref, yhi_ref):
    d = deg_ref[0] + deg_ref[1] + 1.0
    dinv = jnp.where(d > 0, lax.rsqrt(d), 0.0)
    dinv_ref[...] = dinv
    ys = y_ref[...] * dinv
    ylo_ref[...] = ys[:, :FW]
    yhi_ref[...] = ys[:, FW:]


def _tc_prep(deg3, Y):
    return pl.pallas_call(
        _tc_prep_body,
        grid=(N // _BLK,),
        in_specs=[
            pl.BlockSpec((2, _BLK, 1), lambda i: (0, i, 0)),
            pl.BlockSpec((_BLK, IN_DIM), lambda i: (i, 0)),
        ],
        out_specs=[
            pl.BlockSpec((_BLK, 1), lambda i: (i, 0)),
            pl.BlockSpec((_BLK, FW), lambda i: (i, 0)),
            pl.BlockSpec((_BLK, FW), lambda i: (i, 0)),
        ],
        out_shape=[
            jax.ShapeDtypeStruct((N, 1), jnp.float32),
            jax.ShapeDtypeStruct((N, FW), jnp.float32),
            jax.ShapeDtypeStruct((N, FW), jnp.float32),
        ],
    )(deg3, Y)


def _combine_u(u_ref):
    return jnp.concatenate([u_ref[0, 0] + u_ref[1, 0],
                            u_ref[0, 1] + u_ref[1, 1]], axis=1)


def _tc_enc_body(u_ref, ylo_ref, yhi_ref, dinv_ref, w1_ref, b1_ref, wcat_ref,
                 hlo_ref, hhi_ref):
    dinv = dinv_ref[...]
    ys = jnp.concatenate([ylo_ref[...], yhi_ref[...]], axis=1)
    ya = (_combine_u(u_ref) + ys) * dinv
    h = jnp.dot(ya, w1_ref[...], preferred_element_type=jnp.float32) + b1_ref[...]
    h = jnp.maximum(h, 0.0)
    hm = jnp.dot(h, wcat_ref[...], preferred_element_type=jnp.float32)
    hs = hm * dinv
    hlo_ref[...] = hs[:, :FW]
    hhi_ref[...] = hs[:, FW:]


def _tc_enc(U1, Ylo, Yhi, dinv, W1, b1r, Wcat):
    return pl.pallas_call(
        _tc_enc_body,
        grid=(N // _BLK,),
        in_specs=[
            pl.BlockSpec((2, 2, _BLK, FW), lambda i: (0, 0, i, 0)),
            pl.BlockSpec((_BLK, FW), lambda i: (i, 0)),
            pl.BlockSpec((_BLK, FW), lambda i: (i, 0)),
            pl.BlockSpec((_BLK, 1), lambda i: (i, 0)),
            pl.BlockSpec((IN_DIM, HID), lambda i: (0, 0)),
            pl.BlockSpec((1, HID), lambda i: (0, 0)),
            pl.BlockSpec((HID, 2 * LAT), lambda i: (0, 0)),
        ],
        out_specs=[
            pl.BlockSpec((_BLK, FW), lambda i: (i, 0)),
            pl.BlockSpec((_BLK, FW), lambda i: (i, 0)),
        ],
        out_shape=[
            jax.ShapeDtypeStruct((N, FW), jnp.float32),
            jax.ShapeDtypeStruct((N, FW), jnp.float32),
        ],
    )(U1, Ylo, Yhi, dinv, W1, b1r, Wcat)


def _tc_dec_body(u_ref, hlo_ref, hhi_ref, dinv_ref, eps_ref, bmu_ref, blv_ref,
                 wd1_ref, bd1_ref, wd2_ref, bd2_ref, xref_ref,
                 yhat_ref, mu_ref, lv_ref, b_ref):
    dinv = dinv_ref[...]
    hs = jnp.concatenate([hlo_ref[...], hhi_ref[...]], axis=1)
    agg = (_combine_u(u_ref) + hs) * dinv
    mu = agg[:, :LAT] + bmu_ref[...]
    lv = jnp.clip(agg[:, LAT:] + blv_ref[...], -10.0, 10.0)
    z = mu + eps_ref[...] * jnp.exp(0.5 * lv)
    hd = jnp.dot(z, wd1_ref[...], preferred_element_type=jnp.float32) + bd1_ref[...]
    hd = jnp.maximum(hd, 0.0)
    logits = (jnp.dot(hd, wd2_ref[...], preferred_element_type=jnp.float32)
              + bd2_ref[...]) / TAU
    m = jnp.max(logits, axis=1, keepdims=True)
    e = jnp.exp(logits - m)
    bmat = e / jnp.sum(e, axis=1, keepdims=True)
    yhat_ref[...] = jnp.dot(bmat, xref_ref[...], preferred_element_type=jnp.float32)
    mu_ref[...] = mu
    lv_ref[...] = lv
    b_ref[...] = bmat[:, :C]


def _tc_dec(U2, Hlo, Hhi, dinv, eps, bmur, blvr, Wd1, bd1r, Wd2p, bd2p, Xrefp):
    CPAD = LAT
    return pl.pallas_call(
        _tc_dec_body,
        grid=(N // _BLK,),
        in_specs=[
            pl.BlockSpec((2, 2, _BLK, FW), lambda i: (0, 0, i, 0)),
            pl.BlockSpec((_BLK, FW), lambda i: (i, 0)),
            pl.BlockSpec((_BLK, FW), lambda i: (i, 0)),
            pl.BlockSpec((_BLK, 1), lambda i: (i, 0)),
            pl.BlockSpec((_BLK, LAT), lambda i: (i, 0)),
            pl.BlockSpec((1, LAT), lambda i: (0, 0)),
            pl.BlockSpec((1, LAT), lambda i: (0, 0)),
            pl.BlockSpec((LAT, HID), lambda i: (0, 0)),
            pl.BlockSpec((1, HID), lambda i: (0, 0)),
            pl.BlockSpec((HID, CPAD), lambda i: (0, 0)),
            pl.BlockSpec((1, CPAD), lambda i: (0, 0)),
            pl.BlockSpec((CPAD, IN_DIM), lambda i: (0, 0)),
        ],
        out_specs=[
            pl.BlockSpec((_BLK, IN_DIM), lambda i: (i, 0)),
            pl.BlockSpec((_BLK, LAT), lambda i: (i, 0)),
            pl.BlockSpec((_BLK, LAT), lambda i: (i, 0)),
            pl.BlockSpec((_BLK, C), lambda i: (i, 0)),
        ],
        out_shape=[
            jax.ShapeDtypeStruct((N, IN_DIM), jnp.float32),
            jax.ShapeDtypeStruct((N, LAT), jnp.float32),
            jax.ShapeDtypeStruct((N, LAT), jnp.float32),
            jax.ShapeDtypeStruct((N, C), jnp.float32),
        ],
    )(U2, Hlo, Hhi, dinv, eps, bmur, blvr, Wd1, bd1r, Wd2p, bd2p, Xrefp)


def kernel(Y, edge_index, edge_weight, X_ref, W1, b1, Wmu, bmu, Wlv, blv,
           Wd1, bd1, Wd2, bd2):
    f32 = jnp.float32
    src = edge_index[0].astype(jnp.int32)
    dst = edge_index[1].astype(jnp.int32)
    pad = EPAD - E
    zi = jnp.zeros((pad,), jnp.int32)
    srcF = jnp.concatenate([src, zi]).reshape(RPAD, CHUNK)
    dstF = jnp.concatenate([dst, zi]).reshape(RPAD, CHUNK)
    wF = jnp.concatenate([edge_weight, jnp.zeros((pad,), f32)]).reshape(RPAD, CHUNK)

    zeros2 = jnp.zeros((NP_, FW), f32)
    eps = jax.random.normal(jax.random.key(42), (N, LAT), dtype=f32)

    Wcat = jnp.concatenate([Wmu, Wlv], axis=1)
    Wd2p = jnp.concatenate([Wd2, jnp.zeros((HID, LAT - C), f32)], axis=1)
    bd2p = jnp.concatenate([bd2, jnp.full((LAT - C,), -1e30, f32)]).reshape(1, LAT)
    Xrefp = jnp.concatenate([X_ref, jnp.zeros((LAT - C, IN_DIM), f32)], axis=0)

    perm = np.concatenate([
        np.stack([np.arange(16) + g * 32, np.arange(16) + g * 32 + 16],
                 axis=1).ravel()
        for g in range(FW // 32)
    ])

    def bf16p(x):
        return x[:, perm].astype(jnp.bfloat16)

    sc_deg, sc_agg = _sc_kernels()
    deg2 = sc_deg(dstF, wF, jnp.zeros((NP_,), f32))
    dinv, Ylo, Yhi = _tc_prep(deg2.reshape(NC, NP_, 1), Y)
    U1 = sc_agg(srcF, dstF, wF, bf16p(Ylo), bf16p(Yhi), zeros2)
    Hlo, Hhi = _tc_enc(U1, Ylo, Yhi, dinv, W1, b1.reshape(1, HID), Wcat)
    U2 = sc_agg(srcF, dstF, wF, bf16p(Hlo), bf16p(Hhi), zeros2)
    Y_hat, mu, logvar, Bmat = _tc_dec(
        U2, Hlo, Hhi, dinv, eps, bmu.reshape(1, LAT), blv.reshape(1, LAT),
        Wd1, bd1.reshape(1, HID), Wd2p, bd2p, Xrefp)
    return (Y_hat, mu, logvar, Bmat)
